# Initial kernel scaffold; baseline (speedup 1.0000x reference)
import jax
import jax.numpy as jnp
from jax.experimental import pallas as pl


def kernel(features_1, features_2, params):
    raise NotImplementedError("write your pallas kernel here")


# speedup vs baseline: 10.5587x; 10.5587x over previous
import functools

import jax
import jax.numpy as jnp
import numpy as np
from jax.experimental import pallas as pl
from jax.experimental.pallas import tpu as pltpu

K = 10
GEO_C = 64
NUM_LABELS = 12
B, N = 8, 1024
CP = 64
NEG = -1e30
HIGH = jax.lax.Precision.HIGHEST
bf16 = jnp.bfloat16
f32 = jnp.float32


def _amat(a, oc, h):
    P = np.zeros((oc, h), np.float32)
    for hh in range(h):
        for d in range(4):
            P[hh * 4 + d, hh] = 1.0
    return jnp.asarray(P) * jnp.transpose(a).reshape(oc)[:, None]


def _emat(h, oc):
    E = np.zeros((h, oc), np.float32)
    for hh in range(h):
        for d in range(4):
            E[hh, hh * 4 + d] = 1.0
    return jnp.asarray(E)


def _dot(a, b, dims, prec=HIGH):
    return jax.lax.dot_general(a, b, (dims, ((), ())),
                               preferred_element_type=f32, precision=prec)


def _gat_body(oc, h, xin_ref, xxr_ref, xxc_ref, w_ref, amat_ref, e_ref,
              out_ref):
    n = N
    xxr = xxr_ref[0]
    xxc = xxc_ref[0]
    Wb = w_ref[0].astype(bf16)
    Ab = amat_ref[0].astype(bf16)
    E = e_ref[...]

    ctr_b = xin_ref[0].astype(bf16)

    s2 = _dot(ctr_b, ctr_b, (((1,), (1,))), prec=None)
    inner = -2.0 * s2
    pd = (-xxr - inner) - xxc

    iota = jax.lax.broadcasted_iota(jnp.int32, (n, n), 1)
    hws, ahws = [], []
    for _ in range(K):
        mx = jnp.max(pd, axis=1, keepdims=True)
        cand = pd >= mx
        aidx = jnp.min(jnp.where(cand, iota, n), axis=1, keepdims=True)
        ohb = iota == aidx
        oh = ohb.astype(bf16)
        pd = jnp.where(ohb, NEG, pd)
        nb_b = _dot(oh, ctr_b, (((1,), (0,))), prec=None)
        feat = jnp.concatenate([nb_b.astype(bf16), ctr_b], axis=1)
        hw = _dot(feat, Wb, (((1,), (0,))), prec=None)
        ahw = _dot(hw.astype(bf16), Ab, (((1,), (0,))), prec=None)
        hws.append(hw)
        ahws.append(ahw)

    m = ahws[0]
    for j in range(1, K):
        m = jnp.maximum(m, ahws[j])
    es = [jnp.exp(ahws[j] - m) for j in range(K)]
    den = es[0]
    for j in range(1, K):
        den = den + es[j]
    acc = jnp.zeros((n, oc), f32)
    for j in range(K):
        w = es[j] / den
        acc = acc + _dot(w, E, (((1,), (0,)))) * hws[j]
    out = jnp.where(acc >= 0, acc, 0.2 * acc)
    out_ref[0] = out


def _gat_layer(xin, xxr, xxc, W6, A6, E, oc, h):
    G8 = xin.shape[0]
    C2 = W6.shape[1]
    row = lambda i: (i, 0, 0)
    grp = lambda i: (i // 8, 0, 0)
    in_specs = [pl.BlockSpec((1, N, CP), row),
                pl.BlockSpec((1, 1, N), row),
                pl.BlockSpec((1, N, 1), row),
                pl.BlockSpec((1, C2, oc), grp),
                pl.BlockSpec((1, oc, h), grp),
                pl.BlockSpec((h, oc), lambda i: (0, 0))]
    fn = pl.pallas_call(
        functools.partial(_gat_body, oc, h),
        grid=(G8,), in_specs=in_specs,
        out_specs=pl.BlockSpec((1, N, oc), row),
        out_shape=jax.ShapeDtypeStruct((G8, N, oc), f32),
        compiler_params=pltpu.CompilerParams(
            dimension_semantics=("arbitrary",),
            vmem_limit_bytes=110 * 1024 * 1024),
    )
    return fn(xin, xxr, xxc, W6, A6, E)


def _end_body(geo_ref, xyz_ref, sem_ref, endw_ref, out_ref):
    xc = jnp.concatenate([geo_ref[0], xyz_ref[0], sem_ref[0]],
                         axis=1).astype(bf16)
    ew = endw_ref[...].astype(bf16)
    out_ref[0] = _dot(xc, ew, (((1,), (1,))), prec=None)


def _end_layer(x3, end_W):
    F = 128
    in_specs = [
        pl.BlockSpec((1, N, F),
                     lambda i, br=br: ((i // 8) * 24 + br * 8 + (i % 8), 0, 0))
        for br in range(3)
    ] + [pl.BlockSpec((F, 3 * F), lambda i: (0, 0))]
    fn = pl.pallas_call(
        _end_body, grid=(16,), in_specs=in_specs,
        out_specs=pl.BlockSpec((1, N, F), lambda i: (i, 0, 0)),
        out_shape=jax.ShapeDtypeStruct((16, N, F), f32),
        compiler_params=pltpu.CompilerParams(
            dimension_semantics=("arbitrary",),
            vmem_limit_bytes=64 * 1024 * 1024),
    )
    return fn(x3, x3, x3, end_W)


def _head_body(x1_ref, x2_ref, attw_ref, wtt_ref, wbt_ref, bias_ref,
               fc1w_ref, fc1b_ref, fc2w_ref, fc2b_ref,
               score_ref, so1_ref, so2_ref):
    attw = attw_ref[...]

    def att(e):
        rs = jnp.sum(e, axis=0, keepdims=True) / N
        gc = _dot(rs, attw, (((1,), (0,))))
        tg = jnp.tanh(gc)
        s = jax.nn.sigmoid(_dot(e, tg, (((1,), (1,)))))
        rep = _dot(e, s, (((0,), (0,))))
        return s, rep

    sa1, rep1 = att(x1_ref[0])
    sa2, rep2 = att(x2_ref[0])

    parts = [
        _dot(wtt_ref[tt], rep2, (((1,), (0,)))) for tt in range(16)
    ]
    V = jnp.concatenate(parts, axis=1)
    scor = _dot(rep1, V, (((0,), (0,))))
    comb = jnp.concatenate([rep1, rep2], axis=0)
    blk = _dot(comb, wbt_ref[...], (((0,), (0,))))
    srow = jnp.maximum(scor + blk + bias_ref[...], 0.0)
    f1 = jnp.maximum(_dot(srow, fc1w_ref[...], (((1,), (1,)))) + fc1b_ref[...], 0.0)
    sc = jax.nn.sigmoid(_dot(f1, fc2w_ref[...], (((1,), (1,)))) + fc2b_ref[...])
    score_ref[0] = sc
    so1_ref[0] = sa1
    so2_ref[0] = sa2


def _head(emb, att_W, WtT, WbT, bias_row, fc1_W, fc1b_row, fc2w_rep,
          fc2b_row):
    F = 128
    in_specs = [pl.BlockSpec((1, N, F), lambda i: (i, 0, 0)),
                pl.BlockSpec((1, N, F), lambda i: (i + 8, 0, 0)),
                pl.BlockSpec((F, F), lambda i: (0, 0)),
                pl.BlockSpec((16, F, F), lambda i: (0, 0, 0)),
                pl.BlockSpec((2 * F, 16), lambda i: (0, 0)),
                pl.BlockSpec((1, 16), lambda i: (0, 0)),
                pl.BlockSpec((16, 16), lambda i: (0, 0)),
                pl.BlockSpec((1, 16), lambda i: (0, 0)),
                pl.BlockSpec((16, 16), lambda i: (0, 0)),
                pl.BlockSpec((1, 16), lambda i: (0, 0))]
    row = lambda i: (i, 0, 0)
    out_specs = [pl.BlockSpec((1, 1, 16), row),
                 pl.BlockSpec((1, N, 1), row),
                 pl.BlockSpec((1, N, 1), row)]
    out_shape = [jax.ShapeDtypeStruct((B, 1, 16), f32),
                 jax.ShapeDtypeStruct((B, N, 1), f32),
                 jax.ShapeDtypeStruct((B, N, 1), f32)]
    fn = pl.pallas_call(
        _head_body, grid=(B,), in_specs=in_specs, out_specs=out_specs,
        out_shape=out_shape,
        compiler_params=pltpu.CompilerParams(
            dimension_semantics=("arbitrary",),
            vmem_limit_bytes=64 * 1024 * 1024),
    )
    return fn(emb, emb, att_W, WtT, WbT, bias_row, fc1_W, fc1b_row,
              fc2w_rep, fc2b_row)


def _bn1d_ref(x, gamma, beta, eps=1e-5):
    mean = jnp.mean(x, axis=(0, 2), keepdims=True)
    var = jnp.var(x, axis=(0, 2), keepdims=True)
    xn = (x - mean) / jnp.sqrt(var + eps)
    return xn * gamma[None, :, None] + beta[None, :, None]


def _pad_w(W, C):
    Wt, Wb = W[:C], W[C:]
    if C < CP:
        Wt = jnp.pad(Wt, ((0, CP - C), (0, 0)))
        Wb = jnp.pad(Wb, ((0, CP - C), (0, 0)))
    return jnp.concatenate([Wt, Wb], axis=0)


def kernel(features_1, features_2, params):
    branches = [params['geo'], params['center'], params['sem']]
    in_c = [[GEO_C, 64, 64], [3, 64, 64], [NUM_LABELS, 64, 64]]
    ocs = [64, 64, 128]

    xs = []
    for F in (features_1, features_2):
        xs += [F[:, :GEO_C, :], F[:, GEO_C:GEO_C + 3, :],
               F[:, GEO_C + 3:, :]]

    for l in range(3):
        oc, h = ocs[l], ocs[l] // 4
        W6, A6 = [], []
        for pi in range(2):
            for br in range(3):
                p = branches[br][l]
                W6.append(_pad_w(p['W'], in_c[br][l]))
                A6.append(_amat(p['a'], oc, h))
        W6 = jnp.stack(W6)
        A6 = jnp.stack(A6)
        E = _emat(h, oc)

        xin, xxr, xxc = [], [], []
        for x in xs:
            C = x.shape[1]
            xx = jnp.sum(x * x, axis=1, keepdims=True)
            xt = jnp.swapaxes(x, 1, 2)
            if C < CP:
                xt = jnp.pad(xt, ((0, 0), (0, 0), (0, CP - C)))
            xin.append(xt)
            xxr.append(xx)
            xxc.append(jnp.swapaxes(xx, 1, 2))
        xin = jnp.concatenate(xin, axis=0)
        xxr = jnp.concatenate(xxr, axis=0)
        xxc = jnp.concatenate(xxc, axis=0)

        out = _gat_layer(xin, xxr, xxc, W6, A6, E, oc, h)

        nxt = []
        for gi in range(6):
            p = branches[gi % 3][l]
            o = jnp.swapaxes(out[gi * 8:(gi + 1) * 8], 1, 2)
            o = jax.nn.leaky_relu(_bn1d_ref(o, p['gamma'], p['beta']), 0.2)
            nxt.append(o)
        xs = nxt

    x3 = jnp.concatenate([jnp.swapaxes(x, 1, 2) for x in xs], axis=0)
    xend = _end_layer(x3, params['end_W'])
    xe = jnp.swapaxes(xend, 1, 2).reshape(2, 8, 128, N)
    emb = []
    for pi in range(2):
        o = jax.nn.leaky_relu(
            _bn1d_ref(xe[pi], params['end_gamma'], params['end_beta']), 0.2)
        emb.append(jnp.swapaxes(o, 1, 2))
    emb = jnp.concatenate(emb, axis=0)

    WtT = jnp.transpose(params['tn_W'], (2, 0, 1))
    WbT = jnp.transpose(params['tn_Wb'], (1, 0))
    bias_row = jnp.reshape(params['tn_bias'], (1, 16))
    fc1b_row = jnp.reshape(params['fc1_b'], (1, 16))
    fc2w_rep = jnp.tile(params['fc2_W'], (16, 1))
    fc2b_row = jnp.tile(jnp.reshape(params['fc2_b'], (1, 1)), (1, 16))
    score, s1, s2 = _head(emb, params['att_W'], WtT, WbT, bias_row,
                          params['fc1_W'], fc1b_row, fc2w_rep, fc2b_row)
    return (score[:, 0, 0], s1, s2)


# speedup vs baseline: 11.4717x; 1.0865x over previous
import functools

import jax
import jax.numpy as jnp
import numpy as np
from jax.experimental import pallas as pl
from jax.experimental.pallas import tpu as pltpu

K = 10
GEO_C = 64
NUM_LABELS = 12
B, N = 8, 1024
CP = 64
NEG = -1e30
HIGH = jax.lax.Precision.HIGHEST
bf16 = jnp.bfloat16
f32 = jnp.float32


def _amat(a, oc, h):
    P = np.zeros((oc, h), np.float32)
    for hh in range(h):
        for d in range(4):
            P[hh * 4 + d, hh] = 1.0
    return jnp.asarray(P) * jnp.transpose(a).reshape(oc)[:, None]


def _emat(h, oc):
    E = np.zeros((h, oc), np.float32)
    for hh in range(h):
        for d in range(4):
            E[hh, hh * 4 + d] = 1.0
    return jnp.asarray(E)


def _dot(a, b, dims, prec=HIGH):
    return jax.lax.dot_general(a, b, (dims, ((), ())),
                               preferred_element_type=f32, precision=prec)


def _gat_body(oc, h, xin_ref, xxr_ref, xxc_ref, w_ref, amat_ref, e_ref,
              out_ref):
    n = N
    xxr = xxr_ref[0]
    xxc = xxc_ref[0]
    Wb = w_ref[0].astype(bf16)
    Ab = amat_ref[0].astype(bf16)
    E = e_ref[...]

    ctr_b = xin_ref[0].astype(bf16)

    s2 = _dot(ctr_b, ctr_b, (((1,), (1,))), prec=None)
    inner = -2.0 * s2
    pd = (-xxr - inner) - xxc

    iota = jax.lax.broadcasted_iota(jnp.int32, (n, n), 1)
    hws, ahws = [], []
    for _ in range(K):
        aidx = jnp.argmax(pd, axis=1).reshape(n, 1)
        ohb = iota == aidx
        oh = ohb.astype(bf16)
        pd = jnp.where(ohb, NEG, pd)
        nb_b = _dot(oh, ctr_b, (((1,), (0,))), prec=None)
        feat = jnp.concatenate([nb_b.astype(bf16), ctr_b], axis=1)
        hw = _dot(feat, Wb, (((1,), (0,))), prec=None)
        ahw = _dot(hw.astype(bf16), Ab, (((1,), (0,))), prec=None)
        hws.append(hw)
        ahws.append(ahw)

    m = ahws[0]
    for j in range(1, K):
        m = jnp.maximum(m, ahws[j])
    es = [jnp.exp(ahws[j] - m) for j in range(K)]
    den = es[0]
    for j in range(1, K):
        den = den + es[j]
    acc = jnp.zeros((n, oc), f32)
    for j in range(K):
        w = es[j] / den
        acc = acc + _dot(w, E, (((1,), (0,)))) * hws[j]
    out = jnp.where(acc >= 0, acc, 0.2 * acc)
    out_ref[0] = out


def _gat_layer(xin, xxr, xxc, W6, A6, E, oc, h):
    G8 = xin.shape[0]
    C2 = W6.shape[1]
    row = lambda i: (i, 0, 0)
    grp = lambda i: (i // 8, 0, 0)
    in_specs = [pl.BlockSpec((1, N, CP), row),
                pl.BlockSpec((1, 1, N), row),
                pl.BlockSpec((1, N, 1), row),
                pl.BlockSpec((1, C2, oc), grp),
                pl.BlockSpec((1, oc, h), grp),
                pl.BlockSpec((h, oc), lambda i: (0, 0))]
    fn = pl.pallas_call(
        functools.partial(_gat_body, oc, h),
        grid=(G8,), in_specs=in_specs,
        out_specs=pl.BlockSpec((1, N, oc), row),
        out_shape=jax.ShapeDtypeStruct((G8, N, oc), f32),
        compiler_params=pltpu.CompilerParams(
            dimension_semantics=("arbitrary",),
            vmem_limit_bytes=110 * 1024 * 1024),
    )
    return fn(xin, xxr, xxc, W6, A6, E)


def _end_body(geo_ref, xyz_ref, sem_ref, endw_ref, out_ref):
    xc = jnp.concatenate([geo_ref[0], xyz_ref[0], sem_ref[0]],
                         axis=1).astype(bf16)
    ew = endw_ref[...].astype(bf16)
    out_ref[0] = _dot(xc, ew, (((1,), (1,))), prec=None)


def _end_layer(x3, end_W):
    F = 128
    in_specs = [
        pl.BlockSpec((1, N, F),
                     lambda i, br=br: ((i // 8) * 24 + br * 8 + (i % 8), 0, 0))
        for br in range(3)
    ] + [pl.BlockSpec((F, 3 * F), lambda i: (0, 0))]
    fn = pl.pallas_call(
        _end_body, grid=(16,), in_specs=in_specs,
        out_specs=pl.BlockSpec((1, N, F), lambda i: (i, 0, 0)),
        out_shape=jax.ShapeDtypeStruct((16, N, F), f32),
        compiler_params=pltpu.CompilerParams(
            dimension_semantics=("arbitrary",),
            vmem_limit_bytes=64 * 1024 * 1024),
    )
    return fn(x3, x3, x3, end_W)


def _head_body(x1_ref, x2_ref, attw_ref, wtt_ref, wbt_ref, bias_ref,
               fc1w_ref, fc1b_ref, fc2w_ref, fc2b_ref,
               score_ref, so1_ref, so2_ref):
    attw = attw_ref[...]

    def att(e):
        rs = jnp.sum(e, axis=0, keepdims=True) / N
        gc = _dot(rs, attw, (((1,), (0,))))
        tg = jnp.tanh(gc)
        s = jax.nn.sigmoid(_dot(e, tg, (((1,), (1,)))))
        rep = _dot(e, s, (((0,), (0,))))
        return s, rep

    sa1, rep1 = att(x1_ref[0])
    sa2, rep2 = att(x2_ref[0])

    parts = [
        _dot(wtt_ref[tt], rep2, (((1,), (0,)))) for tt in range(16)
    ]
    V = jnp.concatenate(parts, axis=1)
    scor = _dot(rep1, V, (((0,), (0,))))
    comb = jnp.concatenate([rep1, rep2], axis=0)
    blk = _dot(comb, wbt_ref[...], (((0,), (0,))))
    srow = jnp.maximum(scor + blk + bias_ref[...], 0.0)
    f1 = jnp.maximum(_dot(srow, fc1w_ref[...], (((1,), (1,)))) + fc1b_ref[...], 0.0)
    sc = jax.nn.sigmoid(_dot(f1, fc2w_ref[...], (((1,), (1,)))) + fc2b_ref[...])
    score_ref[0] = sc
    so1_ref[0] = sa1
    so2_ref[0] = sa2


def _head(emb, att_W, WtT, WbT, bias_row, fc1_W, fc1b_row, fc2w_rep,
          fc2b_row):
    F = 128
    in_specs = [pl.BlockSpec((1, N, F), lambda i: (i, 0, 0)),
                pl.BlockSpec((1, N, F), lambda i: (i + 8, 0, 0)),
                pl.BlockSpec((F, F), lambda i: (0, 0)),
                pl.BlockSpec((16, F, F), lambda i: (0, 0, 0)),
                pl.BlockSpec((2 * F, 16), lambda i: (0, 0)),
                pl.BlockSpec((1, 16), lambda i: (0, 0)),
                pl.BlockSpec((16, 16), lambda i: (0, 0)),
                pl.BlockSpec((1, 16), lambda i: (0, 0)),
                pl.BlockSpec((16, 16), lambda i: (0, 0)),
                pl.BlockSpec((1, 16), lambda i: (0, 0))]
    row = lambda i: (i, 0, 0)
    out_specs = [pl.BlockSpec((1, 1, 16), row),
                 pl.BlockSpec((1, N, 1), row),
                 pl.BlockSpec((1, N, 1), row)]
    out_shape = [jax.ShapeDtypeStruct((B, 1, 16), f32),
                 jax.ShapeDtypeStruct((B, N, 1), f32),
                 jax.ShapeDtypeStruct((B, N, 1), f32)]
    fn = pl.pallas_call(
        _head_body, grid=(B,), in_specs=in_specs, out_specs=out_specs,
        out_shape=out_shape,
        compiler_params=pltpu.CompilerParams(
            dimension_semantics=("arbitrary",),
            vmem_limit_bytes=64 * 1024 * 1024),
    )
    return fn(emb, emb, att_W, WtT, WbT, bias_row, fc1_W, fc1b_row,
              fc2w_rep, fc2b_row)


def _bn1d_ref(x, gamma, beta, eps=1e-5):
    mean = jnp.mean(x, axis=(0, 2), keepdims=True)
    var = jnp.var(x, axis=(0, 2), keepdims=True)
    xn = (x - mean) / jnp.sqrt(var + eps)
    return xn * gamma[None, :, None] + beta[None, :, None]


def _pad_w(W, C):
    Wt, Wb = W[:C], W[C:]
    if C < CP:
        Wt = jnp.pad(Wt, ((0, CP - C), (0, 0)))
        Wb = jnp.pad(Wb, ((0, CP - C), (0, 0)))
    return jnp.concatenate([Wt, Wb], axis=0)


def kernel(features_1, features_2, params):
    branches = [params['geo'], params['center'], params['sem']]
    in_c = [[GEO_C, 64, 64], [3, 64, 64], [NUM_LABELS, 64, 64]]
    ocs = [64, 64, 128]

    xs = []
    for F in (features_1, features_2):
        xs += [F[:, :GEO_C, :], F[:, GEO_C:GEO_C + 3, :],
               F[:, GEO_C + 3:, :]]

    for l in range(3):
        oc, h = ocs[l], ocs[l] // 4
        W6, A6 = [], []
        for pi in range(2):
            for br in range(3):
                p = branches[br][l]
                W6.append(_pad_w(p['W'], in_c[br][l]))
                A6.append(_amat(p['a'], oc, h))
        W6 = jnp.stack(W6)
        A6 = jnp.stack(A6)
        E = _emat(h, oc)

        xin, xxr, xxc = [], [], []
        for x in xs:
            C = x.shape[1]
            xx = jnp.sum(x * x, axis=1, keepdims=True)
            xt = jnp.swapaxes(x, 1, 2)
            if C < CP:
                xt = jnp.pad(xt, ((0, 0), (0, 0), (0, CP - C)))
            xin.append(xt)
            xxr.append(xx)
            xxc.append(jnp.swapaxes(xx, 1, 2))
        xin = jnp.concatenate(xin, axis=0)
        xxr = jnp.concatenate(xxr, axis=0)
        xxc = jnp.concatenate(xxc, axis=0)

        out = _gat_layer(xin, xxr, xxc, W6, A6, E, oc, h)

        nxt = []
        for gi in range(6):
            p = branches[gi % 3][l]
            o = jnp.swapaxes(out[gi * 8:(gi + 1) * 8], 1, 2)
            o = jax.nn.leaky_relu(_bn1d_ref(o, p['gamma'], p['beta']), 0.2)
            nxt.append(o)
        xs = nxt

    x3 = jnp.concatenate([jnp.swapaxes(x, 1, 2) for x in xs], axis=0)
    xend = _end_layer(x3, params['end_W'])
    xe = jnp.swapaxes(xend, 1, 2).reshape(2, 8, 128, N)
    emb = []
    for pi in range(2):
        o = jax.nn.leaky_relu(
            _bn1d_ref(xe[pi], params['end_gamma'], params['end_beta']), 0.2)
        emb.append(jnp.swapaxes(o, 1, 2))
    emb = jnp.concatenate(emb, axis=0)

    WtT = jnp.transpose(params['tn_W'], (2, 0, 1))
    WbT = jnp.transpose(params['tn_Wb'], (1, 0))
    bias_row = jnp.reshape(params['tn_bias'], (1, 16))
    fc1b_row = jnp.reshape(params['fc1_b'], (1, 16))
    fc2w_rep = jnp.tile(params['fc2_W'], (16, 1))
    fc2b_row = jnp.tile(jnp.reshape(params['fc2_b'], (1, 1)), (1, 16))
    score, s1, s2 = _head(emb, params['att_W'], WtT, WbT, bias_row,
                          params['fc1_W'], fc1b_row, fc2w_rep, fc2b_row)
    return (score[:, 0, 0], s1, s2)
